# baseline (speedup 1.0000x reference)
import functools

import jax
import jax.numpy as jnp
import numpy as np
from jax import lax
from jax.experimental import pallas as pl
from jax.experimental.pallas import tpu as pltpu
from jax.experimental.pallas import tpu_sc as plsc

_NC = 2
_NS = 16
_NW = _NC * _NS
_LANES = 16
_CH = 96
_SB = 48
_NBUF = 4
_NPH = 3


def _matmul_relu_bf16(feature, Wp):
    n, f = feature.shape
    d = Wp.shape[1]
    blk = 1000

    def body(f_ref, w_ref, o_ref):
        o_ref[...] = jnp.maximum(
            jnp.dot(f_ref[...], w_ref[...], preferred_element_type=jnp.float32),
            0.0,
        ).astype(jnp.bfloat16)

    return pl.pallas_call(
        body,
        grid=(n // blk,),
        in_specs=[
            pl.BlockSpec((blk, f), lambda i: (i, 0)),
            pl.BlockSpec((f, d), lambda i: (0, 0)),
        ],
        out_specs=pl.BlockSpec((blk, d), lambda i: (i, 0)),
        out_shape=jax.ShapeDtypeStruct((n, d), jnp.bfloat16),
    )(feature, Wp)


def _combine_bias(partials, b2d):
    nc, n, d = partials.shape
    blk = 1000

    def body(p_ref, b_ref, o_ref):
        o_ref[...] = p_ref[0] + p_ref[1] + b_ref[...]

    return pl.pallas_call(
        body,
        grid=(n // blk,),
        in_specs=[
            pl.BlockSpec((nc, blk, d), lambda i: (0, i, 0)),
            pl.BlockSpec((1, d), lambda i: (0, 0)),
        ],
        out_specs=pl.BlockSpec((blk, d), lambda i: (i, 0)),
        out_shape=jax.ShapeDtypeStruct((n, d), jnp.float32),
    )(partials, b2d)


def _make_spmm(n_nodes, e_pad, d):
    dw = d // 2
    chunks_per_tile = e_pad // (_NW * _CH)
    ept = chunks_per_tile * _CH
    cpp = chunks_per_tile // _NPH
    epp = cpp * _CH
    iters = cpp // _NBUF
    row_chunks_full = n_nodes // _SB
    row_rem = n_nodes - row_chunks_full * _SB
    row_chunks = row_chunks_full + (1 if row_rem else 0)
    row_iters = (row_chunks + _NS - 1) // _NS

    mesh = plsc.VectorSubcoreMesh(core_axis_name="c", subcore_axis_name="s")

    @functools.partial(
        pl.kernel,
        mesh=mesh,
        compiler_params=pltpu.CompilerParams(use_tc_tiling_on_sc=False),
        out_type=jax.ShapeDtypeStruct((_NC, n_nodes, d), jnp.float32),
        scratch_types=[
            pltpu.VMEM((epp,), jnp.int32),
            pltpu.VMEM((2 * cpp, _SB), jnp.int32),
            pltpu.VMEM((epp,), jnp.float32),
            pltpu.VMEM((_CH, dw), jnp.int32),
            pltpu.VMEM((_CH, dw), jnp.int32),
            pltpu.VMEM((_CH, dw), jnp.int32),
            pltpu.VMEM((_CH, dw), jnp.int32),
            pltpu.VMEM((_SB, d), jnp.float32),
            pltpu.VMEM((_SB, d), jnp.float32),
            pltpu.VMEM_SHARED((n_nodes, d), jnp.float32),
            pltpu.SemaphoreType.DMA,
            pltpu.SemaphoreType.DMA,
            pltpu.SemaphoreType.DMA,
            pltpu.SemaphoreType.DMA,
            pltpu.SemaphoreType.DMA,
            pltpu.SemaphoreType.DMA,
        ],
    )
    def spmm(support_hbm, src_hbm, dst_hbm, ew_hbm, out_hbm,
             src_v, dst_v, ew_v, g0, g1, g2, g3, s0, s1, acc_sh,
             gsem0, gsem1, gsem2, gsem3, ssem0, ssem1):
        cid = lax.axis_index("c")
        sid = lax.axis_index("s")
        wid = cid * _NS + sid
        gbufs = [(g0, gsem0), (g1, gsem1), (g2, gsem2), (g3, gsem3)]
        sbufs = [(s0, ssem0), (s1, ssem1)]

        def zrow(j, carry):
            for k in range(d // _LANES):
                s0[j, pl.ds(k * _LANES, _LANES)] = jnp.zeros(
                    (_LANES,), jnp.float32)
            return carry

        lax.fori_loop(0, _SB, zrow, 0)
        for i in range(row_iters):
            j = sid + i * _NS
            r0 = pl.multiple_of(j * _SB, _SB)

            @pl.when(j < row_chunks_full)
            def _():
                pltpu.sync_copy(s0, acc_sh.at[pl.ds(r0, _SB)])

            if row_rem:
                @pl.when(j == row_chunks_full)
                def _():
                    pltpu.sync_copy(
                        s0.at[pl.ds(0, row_rem)],
                        acc_sh.at[pl.ds(row_chunks_full * _SB, row_rem)])
        plsc.subcore_barrier()

        def gather(c, rows, sem):
            return pltpu.async_copy(
                support_hbm.at[src_v.at[pl.ds(c * _CH, _CH)]], rows, sem)

        def gwait(c, rows, sem):
            pltpu.make_async_copy(
                support_hbm.at[src_v.at[pl.ds(c * _CH, _CH)]], rows, sem
            ).wait()

        def scatter(bi, stage, sem):
            return pltpu.async_copy(stage, acc_sh.at[dst_v.at[bi]], sem,
                                    add=True)

        def swait(stage, sem):
            pltpu.make_async_copy(stage, acc_sh.at[dst_v.at[0]], sem).wait()

        bcast_dnums = lax.GatherDimensionNumbers(
            offset_dims=(), collapsed_slice_dims=(0,), start_index_map=(0,))
        himask = jnp.int32(-65536)

        def scale_half(c, h, rows, stage):
            def grp(g16, carry):
                e0 = c * _CH + h * _SB + g16 * _LANES
                eww = ew_v[pl.ds(e0, _LANES)]
                for jj in range(_LANES):
                    bw = lax.gather(
                        eww, jnp.full((_LANES, 1), jj, jnp.int32),
                        bcast_dnums, slice_sizes=(1,),
                        mode=lax.GatherScatterMode.PROMISE_IN_BOUNDS)
                    j = h * _SB + g16 * _LANES + jj
                    jr = g16 * _LANES + jj
                    for g in range(dw // _LANES):
                        w16 = rows[j, pl.ds(g * _LANES, _LANES)]
                        lo = lax.bitcast_convert_type(w16 << 16, jnp.float32)
                        hi = lax.bitcast_convert_type(w16 & himask, jnp.float32)
                        stage[jr, pl.ds(g * _LANES, _LANES)] = lo * bw
                        stage[jr, pl.ds(dw + g * _LANES, _LANES)] = hi * bw
                return carry

            lax.fori_loop(0, _SB // _LANES, grp, 0)

        def phase_body(phase, pcarry):
            ebase = pl.multiple_of(wid * ept + phase * epp, 8)
            pltpu.sync_copy(src_hbm.at[pl.ds(ebase, epp)], src_v)
            pltpu.sync_copy(
                dst_hbm.at[pl.ds(2 * (wid * chunks_per_tile + phase * cpp),
                                 2 * cpp)], dst_v)
            pltpu.sync_copy(ew_hbm.at[pl.ds(ebase, epp)], ew_v)
            for p in range(_NBUF):
                gather(p, gbufs[p][0], gbufs[p][1])

            def body(i4, carry):
                for p in range(_NBUF):
                    c = _NBUF * i4 + p
                    gwait(c, gbufs[p][0], gbufs[p][1])
                    for h in range(2):
                        stage, sem = sbufs[h]
                        if p == 0:
                            @pl.when(i4 > 0)
                            def _():
                                swait(stage, sem)
                        else:
                            swait(stage, sem)
                        scale_half(c, h, gbufs[p][0], stage)
                        scatter(2 * c + h, stage, sem)

                    @pl.when(i4 < iters - 1)
                    def _():
                        gather(c + _NBUF, gbufs[p][0], gbufs[p][1])

                return carry

            lax.fori_loop(0, iters, body, 0)
            swait(s0, ssem0)
            swait(s1, ssem1)
            return pcarry

        lax.fori_loop(0, _NPH, phase_body, 0)
        plsc.subcore_barrier()

        for i in range(row_iters):
            j = sid + i * _NS
            r0 = pl.multiple_of(j * _SB, _SB)

            @pl.when(j < row_chunks_full)
            def _():
                pltpu.sync_copy(acc_sh.at[pl.ds(r0, _SB)],
                                out_hbm.at[cid, pl.ds(r0, _SB)])

            if row_rem:
                @pl.when(j == row_chunks_full)
                def _():
                    rr = row_chunks_full * _SB
                    pltpu.sync_copy(acc_sh.at[pl.ds(rr, row_rem)],
                                    out_hbm.at[cid, pl.ds(rr, row_rem)])

    return spmm


def kernel(feature, edge_index, edge_weight, W, b):
    n, f = feature.shape
    d = W.shape[1]
    dw = d // 2
    e = edge_weight.shape[0]

    perm = np.stack([np.arange(dw), np.arange(dw) + dw], axis=1).ravel()
    support = _matmul_relu_bf16(feature, W[:, perm])
    support_i = lax.bitcast_convert_type(
        support.reshape(n, dw, 2), jnp.int32)

    grain = _NW * _CH
    e_pad = ((e + grain - 1) // grain) * grain
    while (e_pad // grain) % (_NPH * _NBUF):
        e_pad += grain
    pad = e_pad - e
    spread = jnp.arange(pad, dtype=jnp.int32) % n
    src = jnp.concatenate([edge_index[0], spread])
    dst = jnp.concatenate([edge_index[1], spread])
    ew = jnp.pad(edge_weight, (0, pad))
    dst2d = dst.reshape(e_pad // _SB, _SB)

    partials = _make_spmm(n, e_pad, d)(support_i, src, dst2d, ew)
    return _combine_bias(partials, b.reshape(1, d))


# speedup vs baseline: 2.0259x; 2.0259x over previous
import functools

import jax
import jax.numpy as jnp
from jax import lax
from jax.experimental import pallas as pl
from jax.experimental.pallas import tpu as pltpu
from jax.experimental.pallas import tpu_sc as plsc

_NC = 2
_NS = 16
_NW = _NC * _NS
_LANES = 16
_CH = 64
_NBUF = 5
_NPH = 8


def _matmul_relu(feature, W):
    n, f = feature.shape
    d = W.shape[1]
    blk = 1000

    def body(f_ref, w_ref, o_ref):
        o_ref[...] = jnp.maximum(
            jnp.dot(f_ref[...], w_ref[...], preferred_element_type=jnp.float32),
            0.0,
        )

    return pl.pallas_call(
        body,
        grid=(n // blk,),
        in_specs=[
            pl.BlockSpec((blk, f), lambda i: (i, 0)),
            pl.BlockSpec((f, d), lambda i: (0, 0)),
        ],
        out_specs=pl.BlockSpec((blk, d), lambda i: (i, 0)),
        out_shape=jax.ShapeDtypeStruct((n, d), jnp.float32),
    )(feature, W)


def _combine_bias(partials, b2d):
    nc, n, d = partials.shape
    blk = 1000

    def body(p_ref, b_ref, o_ref):
        o_ref[...] = p_ref[0] + p_ref[1] + b_ref[...]

    return pl.pallas_call(
        body,
        grid=(n // blk,),
        in_specs=[
            pl.BlockSpec((nc, blk, d), lambda i: (0, i, 0)),
            pl.BlockSpec((1, d), lambda i: (0, 0)),
        ],
        out_specs=pl.BlockSpec((blk, d), lambda i: (i, 0)),
        out_shape=jax.ShapeDtypeStruct((n, d), jnp.float32),
    )(partials, b2d)


def _make_spmm(n_nodes, e_pad, d):
    chunks_per_tile = e_pad // (_NW * _CH)
    ept = chunks_per_tile * _CH
    cpp = chunks_per_tile // _NPH
    epp = cpp * _CH
    iters = cpp // _NBUF
    rc = 2 * _CH
    row_chunks_full = n_nodes // rc
    row_rem = n_nodes - row_chunks_full * rc
    row_chunks = row_chunks_full + (1 if row_rem else 0)
    row_iters = (row_chunks + _NS - 1) // _NS

    mesh = plsc.VectorSubcoreMesh(core_axis_name="c", subcore_axis_name="s")

    @functools.partial(
        pl.kernel,
        mesh=mesh,
        out_type=jax.ShapeDtypeStruct((_NC, n_nodes, d), jnp.float32),
        scratch_types=[
            pltpu.VMEM((epp,), jnp.int32),
            pltpu.VMEM((cpp, _CH), jnp.int32),
            pltpu.VMEM((epp,), jnp.float32),
            pltpu.VMEM((_CH, d), jnp.float32),
            pltpu.VMEM((_CH, d), jnp.float32),
            pltpu.VMEM((_CH, d), jnp.float32),
            pltpu.VMEM((_CH, d), jnp.float32),
            pltpu.VMEM((_CH, d), jnp.float32),
            pltpu.VMEM_SHARED((n_nodes, d), jnp.float32),
            pltpu.SemaphoreType.DMA,
            pltpu.SemaphoreType.DMA,
            pltpu.SemaphoreType.DMA,
            pltpu.SemaphoreType.DMA,
            pltpu.SemaphoreType.DMA,
            pltpu.SemaphoreType.DMA,
            pltpu.SemaphoreType.DMA,
            pltpu.SemaphoreType.DMA,
            pltpu.SemaphoreType.DMA,
            pltpu.SemaphoreType.DMA,
        ],
    )
    def spmm(support_hbm, src_hbm, dst_hbm, ew_hbm, out_hbm,
             src_v, dst_v, ew_v, r0, r1, r2, r3, r4, acc_sh,
             gs0, gs1, gs2, gs3, gs4, ss0, ss1, ss2, ss3, ss4):
        cid = lax.axis_index("c")
        sid = lax.axis_index("s")
        wid = cid * _NS + sid
        rbufs = [r0, r1, r2, r3, r4]
        gsems = [gs0, gs1, gs2, gs3, gs4]
        ssems = [ss0, ss1, ss2, ss3, ss4]

        def zrow(j, carry):
            for k in range(d // _LANES):
                r0[j, pl.ds(k * _LANES, _LANES)] = jnp.zeros(
                    (_LANES,), jnp.float32)
                r1[j, pl.ds(k * _LANES, _LANES)] = jnp.zeros(
                    (_LANES,), jnp.float32)
            return carry

        lax.fori_loop(0, _CH, zrow, 0)
        for i in range(row_iters):
            j = sid + i * _NS
            rr0 = pl.multiple_of(j * rc, rc)

            @pl.when(j < row_chunks_full)
            def _():
                pltpu.sync_copy(r0, acc_sh.at[pl.ds(rr0, _CH)])
                pltpu.sync_copy(r1, acc_sh.at[pl.ds(rr0 + _CH, _CH)])

            if row_rem:
                @pl.when(j == row_chunks_full)
                def _():
                    pltpu.sync_copy(
                        r0.at[pl.ds(0, row_rem)],
                        acc_sh.at[pl.ds(row_chunks_full * rc, row_rem)])
        plsc.subcore_barrier()

        def gather(c, rows, sem):
            return pltpu.async_copy(
                support_hbm.at[src_v.at[pl.ds(c * _CH, _CH)]], rows, sem)

        def gwait(c, rows, sem):
            pltpu.make_async_copy(
                support_hbm.at[src_v.at[pl.ds(c * _CH, _CH)]], rows, sem
            ).wait()

        def scatter(c, rows, sem):
            return pltpu.async_copy(rows, acc_sh.at[dst_v.at[c]], sem,
                                    add=True)

        def swait(rows, sem):
            pltpu.make_async_copy(rows, acc_sh.at[dst_v.at[0]], sem).wait()

        bcast_dnums = lax.GatherDimensionNumbers(
            offset_dims=(), collapsed_slice_dims=(0,), start_index_map=(0,))

        def scale(c, rows):
            def grp(g, carry):
                eww = ew_v[pl.ds(c * _CH + g * _LANES, _LANES)]
                for jj in range(_LANES):
                    bw = lax.gather(
                        eww, jnp.full((_LANES, 1), jj, jnp.int32),
                        bcast_dnums, slice_sizes=(1,),
                        mode=lax.GatherScatterMode.PROMISE_IN_BOUNDS)
                    j = g * _LANES + jj
                    for k in range(d // _LANES):
                        sl = pl.ds(k * _LANES, _LANES)
                        rows[j, sl] = rows[j, sl] * bw
                return carry

            lax.fori_loop(0, _CH // _LANES, grp, 0)

        def phase_body(phase, pcarry):
            ebase = pl.multiple_of(wid * ept + phase * epp, 8)
            pltpu.sync_copy(src_hbm.at[pl.ds(ebase, epp)], src_v)
            pltpu.sync_copy(dst_hbm.at[wid * _NPH + phase], dst_v)
            pltpu.sync_copy(ew_hbm.at[pl.ds(ebase, epp)], ew_v)
            for q in range(_NBUF - 1):
                gather(q, rbufs[q], gsems[q])

            def body(i5, carry):
                for q in range(_NBUF):
                    c = _NBUF * i5 + q
                    gwait(c, rbufs[q], gsems[q])
                    scale(c, rbufs[q])
                    scatter(c, rbufs[q], ssems[q])
                    pn = (q + _NBUF - 1) % _NBUF
                    if q == 0:
                        @pl.when(i5 > 0)
                        def _():
                            swait(rbufs[pn], ssems[pn])
                    else:
                        swait(rbufs[pn], ssems[pn])

                    @pl.when(c + _NBUF - 1 < cpp)
                    def _():
                        gather(c + _NBUF - 1, rbufs[pn], gsems[pn])
                return carry

            lax.fori_loop(0, iters, body, 0)
            swait(rbufs[_NBUF - 1], ssems[_NBUF - 1])
            return pcarry

        lax.fori_loop(0, _NPH, phase_body, 0)
        plsc.subcore_barrier()

        for i in range(row_iters):
            j = sid + i * _NS
            rr0 = pl.multiple_of(j * rc, rc)

            @pl.when(j < row_chunks_full)
            def _():
                pltpu.sync_copy(acc_sh.at[pl.ds(rr0, rc)],
                                out_hbm.at[cid, pl.ds(rr0, rc)])

            if row_rem:
                @pl.when(j == row_chunks_full)
                def _():
                    rrr = row_chunks_full * rc
                    pltpu.sync_copy(acc_sh.at[pl.ds(rrr, row_rem)],
                                    out_hbm.at[cid, pl.ds(rrr, row_rem)])

    return spmm


def kernel(feature, edge_index, edge_weight, W, b):
    n, f = feature.shape
    d = W.shape[1]
    e = edge_weight.shape[0]

    support = _matmul_relu(feature, W)

    grain = _NW * _CH
    e_pad = ((e + grain - 1) // grain) * grain
    while (e_pad // grain) % (_NPH * _NBUF):
        e_pad += grain
    pad = e_pad - e
    spread = jnp.arange(pad, dtype=jnp.int32) % n
    src = jnp.concatenate([edge_index[0], spread])
    dst = jnp.concatenate([edge_index[1], spread])
    ew = jnp.pad(edge_weight, (0, pad))
    cpp = e_pad // (_NW * _CH * _NPH)
    dst3d = dst.reshape(_NW * _NPH, cpp, _CH)

    partials = _make_spmm(n, e_pad, d)(support, src, dst3d, ew)
    return _combine_bias(partials, b.reshape(1, d))
